# Initial kernel scaffold; baseline (speedup 1.0000x reference)
import jax
import jax.numpy as jnp
from jax.experimental import pallas as pl


def kernel(inputs, emb_weight):
    raise NotImplementedError("write your pallas kernel here")


# speedup vs baseline: 4.6070x; 4.6070x over previous
import functools

import jax
import jax.numpy as jnp
from jax import lax
from jax.experimental import pallas as pl
from jax.experimental.pallas import tpu as pltpu
from jax.experimental.pallas import tpu_sc as plsc

_K = 8192
_D = 256
_B = 16
_T = 1024
_N = _B * _T
_TBLK = 256
_NBLK = _N // _TBLK
_TPB = _T // _TBLK


def _vq_tc_body(x_ref, e_ref, idx_ref, loss_ref, perp_ref,
                counts_ref, loss_sum_ref):
    m = pl.program_id(0)
    xblk = x_ref[0]
    e = e_ref[...]
    c = jnp.sum(e * e, axis=1, keepdims=True)
    s = lax.dot_general(e, xblk, (((1,), (0,)), ((), ())),
                        preferred_element_type=jnp.float32,
                        precision=lax.Precision.HIGHEST)
    colsq = jnp.sum(xblk * xblk, axis=0, keepdims=True)
    dist = (colsq + c) - 2.0 * s
    mins = jnp.min(dist, axis=0)
    eq = dist == mins[None, :]
    kio = lax.broadcasted_iota(jnp.int32, dist.shape, 0)
    idx = jnp.min(jnp.where(eq, kio, jnp.int32(2 ** 30)), axis=0)
    idx_ref[0, 0, :] = idx

    @pl.when(m == 0)
    def _():
        counts_ref[...] = jnp.zeros_like(counts_ref)
        loss_sum_ref[0] = 0.0

    onehot = (kio == idx[None, :]).astype(jnp.float32)
    counts_ref[...] += jnp.sum(onehot, axis=1, keepdims=True)
    loss_sum_ref[0] += jnp.sum(mins)

    @pl.when(m == _NBLK - 1)
    def _():
        loss_ref[...] = jnp.reshape(1.25 * loss_sum_ref[0] / (_N * _D), (1, 1))
        p = counts_ref[...] * (1.0 / _N)
        ent = -jnp.sum(p * jnp.log(p + 1e-10))
        perp_ref[...] = jnp.reshape(jnp.exp(ent), (1, 1))


def _vq_tc(inputs, emb_weight):
    return pl.pallas_call(
        _vq_tc_body,
        grid=(_NBLK,),
        in_specs=[
            pl.BlockSpec((1, _D, _TBLK), lambda m: (m // _TPB, 0, m % _TPB)),
            pl.BlockSpec((_K, _D), lambda m: (0, 0)),
        ],
        out_specs=[
            pl.BlockSpec((1, 1, _TBLK), lambda m: (m, 0, 0)),
            pl.BlockSpec((1, 1), lambda m: (0, 0)),
            pl.BlockSpec((1, 1), lambda m: (0, 0)),
        ],
        out_shape=[
            jax.ShapeDtypeStruct((_NBLK, 1, _TBLK), jnp.int32),
            jax.ShapeDtypeStruct((1, 1), jnp.float32),
            jax.ShapeDtypeStruct((1, 1), jnp.float32),
        ],
        scratch_shapes=[
            pltpu.VMEM((_K, 1), jnp.float32),
            pltpu.SMEM((1,), jnp.float32),
        ],
    )(inputs, emb_weight)


def _sc_gather(emb_weight, idx_flat):
    NC, NS = 2, 16
    NW = NC * NS
    bpw = _N // NW
    CH = 128
    NCH = bpw // CH
    idx2d = idx_flat.reshape(_N // CH, CH)
    mesh = plsc.VectorSubcoreMesh(core_axis_name="c", subcore_axis_name="s")

    @functools.partial(
        pl.kernel,
        mesh=mesh,
        out_type=jax.ShapeDtypeStruct((_N, _D), jnp.float32),
        scratch_types=[
            pltpu.VMEM((NCH, CH), jnp.int32),
            pltpu.VMEM((CH, _D), jnp.float32),
            pltpu.VMEM((CH, _D), jnp.float32),
            pltpu.SemaphoreType.DMA,
            pltpu.SemaphoreType.DMA,
        ],
    )
    def k(e_hbm, idx_hbm, out_hbm, idx_v, rows_a, rows_b, sem_a, sem_b):
        wid = lax.axis_index("s") * NC + lax.axis_index("c")
        row0 = wid * bpw
        pltpu.sync_copy(idx_hbm.at[pl.ds(wid * NCH, NCH)], idx_v)
        bufs = [(rows_a, sem_a), (rows_b, sem_b)]
        copies = [None, None]
        copies[0] = pltpu.async_copy(e_hbm.at[idx_v.at[0]], rows_a, sem_a)
        for ci in range(NCH):
            cur = ci % 2
            if ci + 1 < NCH:
                nxt = (ci + 1) % 2
                copies[nxt] = pltpu.async_copy(
                    e_hbm.at[idx_v.at[ci + 1]], bufs[nxt][0], bufs[nxt][1])
            copies[cur].wait()
            pltpu.sync_copy(bufs[cur][0], out_hbm.at[pl.ds(row0 + ci * CH, CH)])

    return k(emb_weight, idx2d)


def kernel(inputs, emb_weight):
    idx_blk, loss11, perp11 = _vq_tc(inputs, emb_weight)
    idx_flat = idx_blk.reshape(_N)
    quantized = _sc_gather(emb_weight, idx_flat).reshape(_B, _T, _D)
    return (quantized, loss11[0, 0], perp11[0, 0], idx_flat.reshape(_N, 1))


# speedup vs baseline: 9.1951x; 1.9959x over previous
import functools

import jax
import jax.numpy as jnp
from jax import lax
from jax.experimental import pallas as pl
from jax.experimental.pallas import tpu as pltpu
from jax.experimental.pallas import tpu_sc as plsc

_K = 8192
_D = 256
_B = 16
_T = 1024
_N = _B * _T
_TBLK = 256
_NBLK = _N // _TBLK
_TPB = _T // _TBLK


def _vq_tc_body(x_ref, e_ref, idx_ref, loss_ref, perp_ref,
                counts_ref, loss_sum_ref):
    m = pl.program_id(0)
    xblk = x_ref[0]
    e = e_ref[...]
    c = jnp.sum(e * e, axis=1, keepdims=True)
    s = lax.dot_general(e.astype(jnp.bfloat16), xblk.astype(jnp.bfloat16),
                        (((1,), (0,)), ((), ())),
                        preferred_element_type=jnp.float32)
    colsq = jnp.sum(xblk * xblk, axis=0, keepdims=True)
    dist = (colsq + c) - 2.0 * s
    mins = jnp.min(dist, axis=0)
    eq = dist == mins[None, :]
    kio = lax.broadcasted_iota(jnp.int32, dist.shape, 0)
    idx = jnp.min(jnp.where(eq, kio, jnp.int32(2 ** 30)), axis=0)
    idx_ref[0, 0, :] = idx

    @pl.when(m == 0)
    def _():
        counts_ref[...] = jnp.zeros_like(counts_ref)
        loss_sum_ref[0] = 0.0

    onehot = (kio == idx[None, :]).astype(jnp.float32)
    counts_ref[...] += jnp.sum(onehot, axis=1, keepdims=True)
    loss_sum_ref[0] += jnp.sum(mins)

    @pl.when(m == _NBLK - 1)
    def _():
        loss_ref[...] = jnp.reshape(1.25 * loss_sum_ref[0] / (_N * _D), (1, 1))
        p = counts_ref[...] * (1.0 / _N)
        ent = -jnp.sum(p * jnp.log(p + 1e-10))
        perp_ref[...] = jnp.reshape(jnp.exp(ent), (1, 1))


def _vq_tc(inputs, emb_weight):
    return pl.pallas_call(
        _vq_tc_body,
        grid=(_NBLK,),
        in_specs=[
            pl.BlockSpec((1, _D, _TBLK), lambda m: (m // _TPB, 0, m % _TPB)),
            pl.BlockSpec((_K, _D), lambda m: (0, 0)),
        ],
        out_specs=[
            pl.BlockSpec((1, 1, _TBLK), lambda m: (m, 0, 0)),
            pl.BlockSpec((1, 1), lambda m: (0, 0)),
            pl.BlockSpec((1, 1), lambda m: (0, 0)),
        ],
        out_shape=[
            jax.ShapeDtypeStruct((_NBLK, 1, _TBLK), jnp.int32),
            jax.ShapeDtypeStruct((1, 1), jnp.float32),
            jax.ShapeDtypeStruct((1, 1), jnp.float32),
        ],
        scratch_shapes=[
            pltpu.VMEM((_K, 1), jnp.float32),
            pltpu.SMEM((1,), jnp.float32),
        ],
    )(inputs, emb_weight)


def _sc_gather(emb_weight, idx_flat):
    NC, NS = 2, 16
    NW = NC * NS
    bpw = _N // NW
    CH = 128
    NCH = bpw // CH
    idx2d = idx_flat.reshape(_N // CH, CH)
    mesh = plsc.VectorSubcoreMesh(core_axis_name="c", subcore_axis_name="s")

    @functools.partial(
        pl.kernel,
        mesh=mesh,
        out_type=jax.ShapeDtypeStruct((_N, _D), jnp.float32),
        scratch_types=[
            pltpu.VMEM((NCH, CH), jnp.int32),
            pltpu.VMEM((CH, _D), jnp.float32),
            pltpu.VMEM((CH, _D), jnp.float32),
            pltpu.SemaphoreType.DMA,
            pltpu.SemaphoreType.DMA,
        ],
    )
    def k(e_hbm, idx_hbm, out_hbm, idx_v, rows_a, rows_b, sem_a, sem_b):
        wid = lax.axis_index("s") * NC + lax.axis_index("c")
        row0 = wid * bpw
        pltpu.sync_copy(idx_hbm.at[pl.ds(wid * NCH, NCH)], idx_v)
        bufs = [(rows_a, sem_a), (rows_b, sem_b)]
        copies = [None, None]
        copies[0] = pltpu.async_copy(e_hbm.at[idx_v.at[0]], rows_a, sem_a)
        for ci in range(NCH):
            cur = ci % 2
            if ci + 1 < NCH:
                nxt = (ci + 1) % 2
                copies[nxt] = pltpu.async_copy(
                    e_hbm.at[idx_v.at[ci + 1]], bufs[nxt][0], bufs[nxt][1])
            copies[cur].wait()
            pltpu.sync_copy(bufs[cur][0], out_hbm.at[pl.ds(row0 + ci * CH, CH)])

    return k(emb_weight, idx2d)


def kernel(inputs, emb_weight):
    idx_blk, loss11, perp11 = _vq_tc(inputs, emb_weight)
    idx_flat = idx_blk.reshape(_N)
    quantized = _sc_gather(emb_weight, idx_flat).reshape(_B, _T, _D)
    return (quantized, loss11[0, 0], perp11[0, 0], idx_flat.reshape(_N, 1))
